# baseline (speedup 1.0000x reference)
import functools

import jax
import jax.numpy as jnp
from jax import lax
from jax.experimental import pallas as pl
from jax.experimental.pallas import tpu as pltpu
from jax.experimental.pallas import tpu_sc as plsc

N = 10000
NPAD = 10240
DUMMY = N
NC, NS = 2, 16
NW = NC * NS
CH = 128
DEGW = 16
BLK = 512
GRID = NPAD // BLK
ROWS_PER_TILE = NPAD // NS

_SC_PARAMS = dict(
    mesh=plsc.VectorSubcoreMesh(
        core_axis_name="c", subcore_axis_name="s", num_cores=NC, num_subcores=NS
    ),
    compiler_params=pltpu.CompilerParams(use_tc_tiling_on_sc=False),
)


def _make_deg(npt):
    @functools.partial(
        pl.kernel,
        out_type=jax.ShapeDtypeStruct((NC, NPAD, DEGW), jnp.float32),
        scratch_types=[
            pltpu.VMEM((npt, CH), jnp.int32),
            pltpu.VMEM((CH, DEGW), jnp.float32),
            pltpu.VMEM_SHARED((NPAD, DEGW), jnp.float32),
            pltpu.SemaphoreType.DMA,
            pltpu.SemaphoreType.DMA,
            pltpu.SemaphoreType.DMA,
            pltpu.SemaphoreType.DMA,
        ],
        **_SC_PARAMS,
    )
    def deg_kernel(dst_hbm, zeros_hbm, ones_hbm, out_hbm, dst_v, ones_v, acc,
                   dsem0, dsem1, dsem2, dsem3):
        c = lax.axis_index("c")
        s = lax.axis_index("s")
        wid = s * NC + c
        r0 = s * ROWS_PER_TILE
        pltpu.sync_copy(
            zeros_hbm.at[pl.ds(r0, ROWS_PER_TILE)], acc.at[pl.ds(r0, ROWS_PER_TILE)]
        )
        pltpu.sync_copy(ones_hbm, ones_v)
        pltpu.sync_copy(dst_hbm.at[pl.ds(wid * npt, npt)], dst_v)
        plsc.subcore_barrier()

        ssems = (dsem0, dsem1, dsem2, dsem3)

        @pl.loop(0, npt // 4)
        def _(i):
            j0 = i * 4
            for b in range(4):
                @pl.when(j0 > 0)
                def _(b=b):
                    pltpu.make_async_copy(
                        ones_v, acc.at[pl.ds(0, CH)], ssems[b]
                    ).wait()

                pltpu.async_copy(ones_v, acc.at[dst_v.at[j0 + b]], ssems[b],
                                 add=True)

        for b in range(4):
            pltpu.make_async_copy(ones_v, acc.at[pl.ds(0, CH)], ssems[b]).wait()
        plsc.subcore_barrier()
        pltpu.sync_copy(
            acc.at[pl.ds(r0, ROWS_PER_TILE)], out_hbm.at[c, pl.ds(r0, ROWS_PER_TILE)]
        )

    return deg_kernel


def _pipeline4(npt, fire, drain_g, scatter, drain_s):
    assert npt % 4 == 0
    fire(0, 0)
    fire(1, 1)

    @pl.loop(0, npt // 4)
    def _(i):
        j0 = i * 4
        for b in range(4):
            j = j0 + b
            b2 = (b + 2) % 4
            drain_g(b)
            scatter(j, b)
            if b < 2:
                @pl.when(j0 > 0)
                def _(b2=b2):
                    drain_s(b2)
            else:
                drain_s(b2)

            @pl.when(j + 2 < npt)
            def _(j=j, b2=b2):
                fire(j + 2, b2)

    drain_s(2)
    drain_s(3)


def _make_agg1(npt):
    SG = 20
    NSG = npt // SG
    assert npt % SG == 0 and NSG % 2 == 0 and SG % 4 == 0

    @functools.partial(
        pl.kernel,
        out_type=jax.ShapeDtypeStruct((NC, NPAD, 64), jnp.float32),
        scratch_types=[
            pltpu.VMEM((2, SG, CH), jnp.int32),
            pltpu.VMEM((2, SG, CH), jnp.int32),
            pltpu.VMEM((4, CH, 64), jnp.float32),
            pltpu.VMEM_SHARED((NPAD, 64), jnp.float32),
            pltpu.VMEM_SHARED((NPAD, 64), jnp.float32),
            pltpu.SemaphoreType.DMA,
            pltpu.SemaphoreType.DMA,
            pltpu.SemaphoreType.DMA,
            pltpu.SemaphoreType.DMA,
            pltpu.SemaphoreType.DMA,
            pltpu.SemaphoreType.DMA,
            pltpu.SemaphoreType.DMA,
            pltpu.SemaphoreType.DMA,
            pltpu.SemaphoreType.DMA,
            pltpu.SemaphoreType.DMA,
        ],
        **_SC_PARAMS,
    )
    def agg_kernel(y_hbm, src_hbm, dst_hbm, zeros_hbm, out_hbm,
                   src_v, dst_v, rows_v, y_sp, acc,
                   gsem0, gsem1, gsem2, gsem3,
                   ssem0, ssem1, ssem2, ssem3, isem0, isem1):
        c = lax.axis_index("c")
        s = lax.axis_index("s")
        r0 = s * ROWS_PER_TILE
        pltpu.sync_copy(
            zeros_hbm.at[pl.ds(r0, ROWS_PER_TILE)], acc.at[pl.ds(r0, ROWS_PER_TILE)]
        )
        pltpu.sync_copy(
            y_hbm.at[c, pl.ds(r0, ROWS_PER_TILE)], y_sp.at[pl.ds(r0, ROWS_PER_TILE)]
        )

        gsems = (gsem0, gsem1, gsem2, gsem3)
        ssems = (ssem0, ssem1, ssem2, ssem3)
        isems = (isem0, isem1)
        base = s * npt

        def fire_idx(sg, h):
            pltpu.async_copy(src_hbm.at[pl.ds(base + sg * SG, SG)], src_v.at[h],
                             isems[h])
            pltpu.async_copy(dst_hbm.at[pl.ds(base + sg * SG, SG)], dst_v.at[h],
                             isems[h])

        def drain_idx(h):
            pltpu.make_async_copy(src_hbm.at[pl.ds(0, SG)], src_v.at[h],
                                  isems[h]).wait()
            pltpu.make_async_copy(src_hbm.at[pl.ds(0, SG)], dst_v.at[h],
                                  isems[h]).wait()

        fire_idx(0, 0)
        fire_idx(1, 1)
        plsc.subcore_barrier()

        @pl.loop(0, NSG // 2)
        def _(o):
            for h in range(2):
                sg = 2 * o + h
                drain_idx(h)

                def fire(j, b, _h=h):
                    pltpu.async_copy(y_sp.at[src_v.at[_h].at[j]], rows_v.at[b],
                                     gsems[b])

                def drain_g(b):
                    pltpu.make_async_copy(
                        y_sp.at[pl.ds(0, CH)], rows_v.at[b], gsems[b]
                    ).wait()

                def scatter(j, b, _h=h):
                    pltpu.async_copy(rows_v.at[b], acc.at[dst_v.at[_h].at[j]],
                                     ssems[b], add=True)

                def drain_s(b):
                    pltpu.make_async_copy(
                        rows_v.at[b], acc.at[pl.ds(0, CH)], ssems[b]
                    ).wait()

                _pipeline4(SG, fire, drain_g, scatter, drain_s)

                @pl.when(sg + 2 < NSG)
                def _(_sg=sg, _h=h):
                    fire_idx(_sg + 2, _h)

        plsc.subcore_barrier()
        pltpu.sync_copy(
            acc.at[pl.ds(r0, ROWS_PER_TILE)], out_hbm.at[c, pl.ds(r0, ROWS_PER_TILE)]
        )

    return agg_kernel


def _make_agg2(npt):
    assert npt % 8 == 0

    @functools.partial(
        pl.kernel,
        out_type=jax.ShapeDtypeStruct((NC, NPAD, 40), jnp.float32),
        scratch_types=[
            pltpu.VMEM((npt, CH), jnp.int32),
            pltpu.VMEM((npt, CH), jnp.int32),
            pltpu.VMEM((4, CH, 40), jnp.float32),
            pltpu.VMEM_SHARED((NPAD, 40), jnp.float32),
            pltpu.VMEM_SHARED((NPAD, 40), jnp.float32),
            pltpu.SemaphoreType.DMA,
            pltpu.SemaphoreType.DMA,
            pltpu.SemaphoreType.DMA,
            pltpu.SemaphoreType.DMA,
            pltpu.SemaphoreType.DMA,
            pltpu.SemaphoreType.DMA,
            pltpu.SemaphoreType.DMA,
            pltpu.SemaphoreType.DMA,
        ],
        **_SC_PARAMS,
    )
    def agg_kernel(y_hbm, src_hbm, dst_hbm, zeros_hbm, out_hbm,
                   src_v, dst_v, rows_v, y_sp, acc,
                   gsem0, gsem1, gsem2, gsem3, ssem0, ssem1, ssem2, ssem3):
        c = lax.axis_index("c")
        s = lax.axis_index("s")
        wid = s * NC + c
        r0 = s * ROWS_PER_TILE
        pltpu.sync_copy(
            zeros_hbm.at[pl.ds(r0, ROWS_PER_TILE)], acc.at[pl.ds(r0, ROWS_PER_TILE)]
        )
        pltpu.sync_copy(
            y_hbm.at[pl.ds(r0, ROWS_PER_TILE)], y_sp.at[pl.ds(r0, ROWS_PER_TILE)]
        )
        pltpu.sync_copy(src_hbm.at[pl.ds(wid * npt, npt)], src_v)
        pltpu.sync_copy(dst_hbm.at[pl.ds(wid * npt, npt)], dst_v)
        plsc.subcore_barrier()

        gsems = (gsem0, gsem1, gsem2, gsem3)
        ssems = (ssem0, ssem1, ssem2, ssem3)

        def fire(j, b):
            pltpu.async_copy(y_sp.at[src_v.at[j]], rows_v.at[b], gsems[b])

        def drain_g(b):
            pltpu.make_async_copy(
                y_sp.at[pl.ds(0, CH)], rows_v.at[b], gsems[b]
            ).wait()

        def scatter(j, b):
            pltpu.async_copy(rows_v.at[b], acc.at[dst_v.at[j]], ssems[b],
                             add=True)

        def drain_s(b):
            pltpu.make_async_copy(
                rows_v.at[b], acc.at[pl.ds(0, CH)], ssems[b]
            ).wait()

        _pipeline4(npt, fire, drain_g, scatter, drain_s)

        plsc.subcore_barrier()
        pltpu.sync_copy(
            acc.at[pl.ds(r0, ROWS_PER_TILE)], out_hbm.at[c, pl.ds(r0, ROWS_PER_TILE)]
        )

    return agg_kernel


def _t1(d0, d1, xp):
    def body(d0_ref, d1_ref, x_ref, y_ref, dis_ref):
        i = pl.program_id(0)
        deg = d0_ref[:, 0:1] + d1_ref[:, 0:1] + 1.0
        dis = lax.rsqrt(deg)
        row = lax.broadcasted_iota(jnp.int32, (BLK, 1), 0) + i * BLK
        dis = jnp.where(row < N, dis, 0.0)
        disb = jnp.broadcast_to(dis, (BLK, 128))
        dis_ref[...] = disb
        y = x_ref[...] * disb
        y_ref[0] = y[:, :64]
        y_ref[1] = y[:, 64:]

    return pl.pallas_call(
        body,
        grid=(GRID,),
        in_specs=[
            pl.BlockSpec((BLK, DEGW), lambda i: (i, 0)),
            pl.BlockSpec((BLK, DEGW), lambda i: (i, 0)),
            pl.BlockSpec((BLK, 128), lambda i: (i, 0)),
        ],
        out_specs=[
            pl.BlockSpec((2, BLK, 64), lambda i: (0, i, 0)),
            pl.BlockSpec((BLK, 128), lambda i: (i, 0)),
        ],
        out_shape=[
            jax.ShapeDtypeStruct((2, NPAD, 64), jnp.float32),
            jax.ShapeDtypeStruct((NPAD, 128), jnp.float32),
        ],
    )(d0, d1, xp)


def _t2(p, y, disb, W1, b1, W2):
    def body(p_ref, y_ref, dis_ref, w1_ref, b1_ref, w2_ref, y2_ref):
        dis = dis_ref[...]
        agg = jnp.concatenate([p_ref[0] + y_ref[0], p_ref[1] + y_ref[1]], axis=1)
        a = dis * agg
        h = jnp.dot(a, w1_ref[...], preferred_element_type=jnp.float32) + b1_ref[...]
        h = jnp.maximum(h, 0.0)
        z2 = jnp.dot(h, w2_ref[...], preferred_element_type=jnp.float32)
        y2_ref[...] = dis[:, :40] * z2

    return pl.pallas_call(
        body,
        grid=(GRID,),
        in_specs=[
            pl.BlockSpec((2, BLK, 64), lambda i: (0, i, 0)),
            pl.BlockSpec((2, BLK, 64), lambda i: (0, i, 0)),
            pl.BlockSpec((BLK, 128), lambda i: (i, 0)),
            pl.BlockSpec((128, 256), lambda i: (0, 0)),
            pl.BlockSpec((1, 256), lambda i: (0, 0)),
            pl.BlockSpec((256, 40), lambda i: (0, 0)),
        ],
        out_specs=pl.BlockSpec((BLK, 40), lambda i: (i, 0)),
        out_shape=jax.ShapeDtypeStruct((NPAD, 40), jnp.float32),
    )(p, y, disb, W1, b1, W2)


def _t3(q0, q1, y2, disb, b2):
    def body(q0_ref, q1_ref, y2_ref, dis_ref, b2_ref, out_ref):
        t = dis_ref[:, :40] * (q0_ref[...] + q1_ref[...] + y2_ref[...]) + b2_ref[...]
        m = jnp.max(t, axis=1, keepdims=True)
        e = t - m
        out_ref[...] = e - jnp.log(jnp.sum(jnp.exp(e), axis=1, keepdims=True))

    return pl.pallas_call(
        body,
        grid=(GRID,),
        in_specs=[
            pl.BlockSpec((BLK, 40), lambda i: (i, 0)),
            pl.BlockSpec((BLK, 40), lambda i: (i, 0)),
            pl.BlockSpec((BLK, 40), lambda i: (i, 0)),
            pl.BlockSpec((BLK, 128), lambda i: (i, 0)),
            pl.BlockSpec((1, 40), lambda i: (0, 0)),
        ],
        out_specs=pl.BlockSpec((BLK, 40), lambda i: (i, 0)),
        out_shape=jax.ShapeDtypeStruct((NPAD, 40), jnp.float32),
    )(q0, q1, y2, disb, b2)


def kernel(x, edge_index, W1, b1, W2, b2):
    ei = edge_index.astype(jnp.int32)
    E = ei.shape[1]
    nchunks = -(-E // (CH * NW * 10)) * NW * 10
    EPAD = nchunks * CH
    pad = EPAD - E
    padv = jnp.full((pad,), DUMMY, jnp.int32)
    src = jnp.concatenate([ei[0], padv]).reshape(-1, CH)
    dst = jnp.concatenate([ei[1], padv]).reshape(-1, CH)

    xp = jnp.pad(x, ((0, NPAD - N), (0, 0)))
    z16 = jnp.zeros((NPAD, DEGW), jnp.float32)
    z64 = jnp.zeros((NPAD, 64), jnp.float32)
    z40 = jnp.zeros((NPAD, 40), jnp.float32)
    ones16 = jnp.ones((CH, DEGW), jnp.float32)

    degp = _make_deg(nchunks // NW)(dst, z16, ones16)
    y, disb = _t1(degp[0], degp[1], xp)
    p = _make_agg1(nchunks // NS)(y, src, dst, z64)
    y2 = _t2(p, y, disb, W1, b1.reshape(1, -1), W2)
    q = _make_agg2(nchunks // NW)(y2, src, dst, z40)
    out = _t3(q[0], q[1], y2, disb, b2.reshape(1, -1))
    return out[:N]


# speedup vs baseline: 1.0191x; 1.0191x over previous
import functools

import jax
import jax.numpy as jnp
from jax import lax
from jax.experimental import pallas as pl
from jax.experimental.pallas import tpu as pltpu
from jax.experimental.pallas import tpu_sc as plsc

N = 10000
NPAD = 10240
DUMMY = N
NC, NS = 2, 16
NW = NC * NS
CH = 128
DEGW = 16
BLK = 512
GRID = NPAD // BLK
ROWS_PER_TILE = NPAD // NS

_SC_PARAMS = dict(
    mesh=plsc.VectorSubcoreMesh(
        core_axis_name="c", subcore_axis_name="s", num_cores=NC, num_subcores=NS
    ),
    compiler_params=pltpu.CompilerParams(use_tc_tiling_on_sc=False),
)


def _make_deg(npt):
    @functools.partial(
        pl.kernel,
        out_type=jax.ShapeDtypeStruct((NC, NPAD, DEGW), jnp.float32),
        scratch_types=[
            pltpu.VMEM((npt, CH), jnp.int32),
            pltpu.VMEM((CH, DEGW), jnp.float32),
            pltpu.VMEM_SHARED((NPAD, DEGW), jnp.float32),
            pltpu.SemaphoreType.DMA,
            pltpu.SemaphoreType.DMA,
            pltpu.SemaphoreType.DMA,
            pltpu.SemaphoreType.DMA,
        ],
        **_SC_PARAMS,
    )
    def deg_kernel(dst_hbm, zeros_hbm, ones_hbm, out_hbm, dst_v, ones_v, acc,
                   dsem0, dsem1, dsem2, dsem3):
        c = lax.axis_index("c")
        s = lax.axis_index("s")
        wid = s * NC + c
        r0 = s * ROWS_PER_TILE
        pltpu.sync_copy(
            zeros_hbm.at[pl.ds(r0, ROWS_PER_TILE)], acc.at[pl.ds(r0, ROWS_PER_TILE)]
        )
        pltpu.sync_copy(ones_hbm, ones_v)
        pltpu.sync_copy(dst_hbm.at[pl.ds(wid * npt, npt)], dst_v)
        plsc.subcore_barrier()

        ssems = (dsem0, dsem1, dsem2, dsem3)

        @pl.loop(0, npt // 4)
        def _(i):
            j0 = i * 4
            for b in range(4):
                @pl.when(j0 > 0)
                def _(b=b):
                    pltpu.make_async_copy(
                        ones_v, acc.at[pl.ds(0, CH)], ssems[b]
                    ).wait()

                pltpu.async_copy(ones_v, acc.at[dst_v.at[j0 + b]], ssems[b],
                                 add=True)

        for b in range(4):
            pltpu.make_async_copy(ones_v, acc.at[pl.ds(0, CH)], ssems[b]).wait()
        plsc.subcore_barrier()
        pltpu.sync_copy(
            acc.at[pl.ds(r0, ROWS_PER_TILE)], out_hbm.at[c, pl.ds(r0, ROWS_PER_TILE)]
        )

    return deg_kernel


def _pipeline4(npt, fire, drain_g, scatter, drain_s):
    assert npt % 4 == 0
    fire(0, 0)
    fire(1, 1)

    @pl.loop(0, npt // 4)
    def _(i):
        j0 = i * 4
        for b in range(4):
            j = j0 + b
            b2 = (b + 2) % 4
            drain_g(b)
            scatter(j, b)
            if b < 2:
                @pl.when(j0 > 0)
                def _(b2=b2):
                    drain_s(b2)
            else:
                drain_s(b2)

            @pl.when(j + 2 < npt)
            def _(j=j, b2=b2):
                fire(j + 2, b2)

    drain_s(2)
    drain_s(3)


def _make_agg1(npt):
    SG = 20
    NSG = npt // SG
    assert npt % SG == 0 and NSG % 2 == 0 and SG % 4 == 0

    @functools.partial(
        pl.kernel,
        out_type=jax.ShapeDtypeStruct((NC, NPAD, 64), jnp.float32),
        scratch_types=[
            pltpu.VMEM((2, SG, CH), jnp.int32),
            pltpu.VMEM((2, SG, CH), jnp.int32),
            pltpu.VMEM((4, CH, 64), jnp.float32),
            pltpu.VMEM_SHARED((NPAD, 64), jnp.float32),
            pltpu.VMEM_SHARED((NPAD, 64), jnp.float32),
            pltpu.SemaphoreType.DMA,
            pltpu.SemaphoreType.DMA,
            pltpu.SemaphoreType.DMA,
            pltpu.SemaphoreType.DMA,
            pltpu.SemaphoreType.DMA,
            pltpu.SemaphoreType.DMA,
            pltpu.SemaphoreType.DMA,
            pltpu.SemaphoreType.DMA,
            pltpu.SemaphoreType.DMA,
            pltpu.SemaphoreType.DMA,
        ],
        **_SC_PARAMS,
    )
    def agg_kernel(y_hbm, src_hbm, dst_hbm, out_hbm,
                   src_v, dst_v, rows_v, y_sp, acc,
                   gsem0, gsem1, gsem2, gsem3,
                   ssem0, ssem1, ssem2, ssem3, isem0, isem1):
        c = lax.axis_index("c")
        s = lax.axis_index("s")
        r0 = s * ROWS_PER_TILE
        pltpu.sync_copy(
            y_hbm.at[c, pl.ds(r0, ROWS_PER_TILE)], acc.at[pl.ds(r0, ROWS_PER_TILE)]
        )
        pltpu.sync_copy(
            y_hbm.at[c, pl.ds(r0, ROWS_PER_TILE)], y_sp.at[pl.ds(r0, ROWS_PER_TILE)]
        )

        gsems = (gsem0, gsem1, gsem2, gsem3)
        ssems = (ssem0, ssem1, ssem2, ssem3)
        isems = (isem0, isem1)
        base = s * npt

        def fire_idx(sg, h):
            pltpu.async_copy(src_hbm.at[pl.ds(base + sg * SG, SG)], src_v.at[h],
                             isems[h])
            pltpu.async_copy(dst_hbm.at[pl.ds(base + sg * SG, SG)], dst_v.at[h],
                             isems[h])

        def drain_idx(h):
            pltpu.make_async_copy(src_hbm.at[pl.ds(0, SG)], src_v.at[h],
                                  isems[h]).wait()
            pltpu.make_async_copy(src_hbm.at[pl.ds(0, SG)], dst_v.at[h],
                                  isems[h]).wait()

        fire_idx(0, 0)
        fire_idx(1, 1)
        plsc.subcore_barrier()

        @pl.loop(0, NSG // 2)
        def _(o):
            for h in range(2):
                sg = 2 * o + h
                drain_idx(h)

                def fire(j, b, _h=h):
                    pltpu.async_copy(y_sp.at[src_v.at[_h].at[j]], rows_v.at[b],
                                     gsems[b])

                def drain_g(b):
                    pltpu.make_async_copy(
                        y_sp.at[pl.ds(0, CH)], rows_v.at[b], gsems[b]
                    ).wait()

                def scatter(j, b, _h=h):
                    pltpu.async_copy(rows_v.at[b], acc.at[dst_v.at[_h].at[j]],
                                     ssems[b], add=True)

                def drain_s(b):
                    pltpu.make_async_copy(
                        rows_v.at[b], acc.at[pl.ds(0, CH)], ssems[b]
                    ).wait()

                _pipeline4(SG, fire, drain_g, scatter, drain_s)

                @pl.when(sg + 2 < NSG)
                def _(_sg=sg, _h=h):
                    fire_idx(_sg + 2, _h)

        plsc.subcore_barrier()
        pltpu.sync_copy(
            acc.at[pl.ds(r0, ROWS_PER_TILE)], out_hbm.at[c, pl.ds(r0, ROWS_PER_TILE)]
        )

    return agg_kernel


def _make_agg2(npt):
    assert npt % 8 == 0

    @functools.partial(
        pl.kernel,
        out_type=jax.ShapeDtypeStruct((NC, NPAD, 40), jnp.float32),
        scratch_types=[
            pltpu.VMEM((npt, CH), jnp.int32),
            pltpu.VMEM((npt, CH), jnp.int32),
            pltpu.VMEM((4, CH, 40), jnp.float32),
            pltpu.VMEM_SHARED((NPAD, 40), jnp.float32),
            pltpu.VMEM_SHARED((NPAD, 40), jnp.float32),
            pltpu.SemaphoreType.DMA,
            pltpu.SemaphoreType.DMA,
            pltpu.SemaphoreType.DMA,
            pltpu.SemaphoreType.DMA,
            pltpu.SemaphoreType.DMA,
            pltpu.SemaphoreType.DMA,
            pltpu.SemaphoreType.DMA,
            pltpu.SemaphoreType.DMA,
        ],
        **_SC_PARAMS,
    )
    def agg_kernel(y_hbm, src_hbm, dst_hbm, zeros_hbm, out_hbm,
                   src_v, dst_v, rows_v, y_sp, acc,
                   gsem0, gsem1, gsem2, gsem3, ssem0, ssem1, ssem2, ssem3):
        c = lax.axis_index("c")
        s = lax.axis_index("s")
        wid = s * NC + c
        r0 = s * ROWS_PER_TILE

        @pl.when(c == 0)
        def _():
            pltpu.sync_copy(
                y_hbm.at[pl.ds(r0, ROWS_PER_TILE)], acc.at[pl.ds(r0, ROWS_PER_TILE)]
            )

        @pl.when(c == 1)
        def _():
            pltpu.sync_copy(
                zeros_hbm.at[pl.ds(r0, ROWS_PER_TILE)],
                acc.at[pl.ds(r0, ROWS_PER_TILE)],
            )

        pltpu.sync_copy(
            y_hbm.at[pl.ds(r0, ROWS_PER_TILE)], y_sp.at[pl.ds(r0, ROWS_PER_TILE)]
        )
        pltpu.sync_copy(src_hbm.at[pl.ds(wid * npt, npt)], src_v)
        pltpu.sync_copy(dst_hbm.at[pl.ds(wid * npt, npt)], dst_v)
        plsc.subcore_barrier()

        gsems = (gsem0, gsem1, gsem2, gsem3)
        ssems = (ssem0, ssem1, ssem2, ssem3)

        def fire(j, b):
            pltpu.async_copy(y_sp.at[src_v.at[j]], rows_v.at[b], gsems[b])

        def drain_g(b):
            pltpu.make_async_copy(
                y_sp.at[pl.ds(0, CH)], rows_v.at[b], gsems[b]
            ).wait()

        def scatter(j, b):
            pltpu.async_copy(rows_v.at[b], acc.at[dst_v.at[j]], ssems[b],
                             add=True)

        def drain_s(b):
            pltpu.make_async_copy(
                rows_v.at[b], acc.at[pl.ds(0, CH)], ssems[b]
            ).wait()

        _pipeline4(npt, fire, drain_g, scatter, drain_s)

        plsc.subcore_barrier()
        pltpu.sync_copy(
            acc.at[pl.ds(r0, ROWS_PER_TILE)], out_hbm.at[c, pl.ds(r0, ROWS_PER_TILE)]
        )

    return agg_kernel


def _t1(d0, d1, xp):
    def body(d0_ref, d1_ref, x_ref, y_ref, dis_ref):
        i = pl.program_id(0)
        deg = d0_ref[:, 0:1] + d1_ref[:, 0:1] + 1.0
        dis = lax.rsqrt(deg)
        row = lax.broadcasted_iota(jnp.int32, (BLK, 1), 0) + i * BLK
        dis = jnp.where(row < N, dis, 0.0)
        disb = jnp.broadcast_to(dis, (BLK, 128))
        dis_ref[...] = disb
        y = x_ref[...] * disb
        y_ref[0] = y[:, :64]
        y_ref[1] = y[:, 64:]

    return pl.pallas_call(
        body,
        grid=(GRID,),
        in_specs=[
            pl.BlockSpec((BLK, DEGW), lambda i: (i, 0)),
            pl.BlockSpec((BLK, DEGW), lambda i: (i, 0)),
            pl.BlockSpec((BLK, 128), lambda i: (i, 0)),
        ],
        out_specs=[
            pl.BlockSpec((2, BLK, 64), lambda i: (0, i, 0)),
            pl.BlockSpec((BLK, 128), lambda i: (i, 0)),
        ],
        out_shape=[
            jax.ShapeDtypeStruct((2, NPAD, 64), jnp.float32),
            jax.ShapeDtypeStruct((NPAD, 128), jnp.float32),
        ],
    )(d0, d1, xp)


def _t2(p, disb, W1, b1, W2):
    def body(p_ref, dis_ref, w1_ref, b1_ref, w2_ref, y2_ref):
        dis = dis_ref[...]
        agg = jnp.concatenate([p_ref[0], p_ref[1]], axis=1)
        a = dis * agg
        h = jnp.dot(a, w1_ref[...], preferred_element_type=jnp.float32) + b1_ref[...]
        h = jnp.maximum(h, 0.0)
        z2 = jnp.dot(h, w2_ref[...], preferred_element_type=jnp.float32)
        y2_ref[...] = dis[:, :40] * z2

    return pl.pallas_call(
        body,
        grid=(GRID,),
        in_specs=[
            pl.BlockSpec((2, BLK, 64), lambda i: (0, i, 0)),
            pl.BlockSpec((BLK, 128), lambda i: (i, 0)),
            pl.BlockSpec((128, 256), lambda i: (0, 0)),
            pl.BlockSpec((1, 256), lambda i: (0, 0)),
            pl.BlockSpec((256, 40), lambda i: (0, 0)),
        ],
        out_specs=pl.BlockSpec((BLK, 40), lambda i: (i, 0)),
        out_shape=jax.ShapeDtypeStruct((NPAD, 40), jnp.float32),
    )(p, disb, W1, b1, W2)


def _t3(q0, q1, disb, b2):
    def body(q0_ref, q1_ref, dis_ref, b2_ref, out_ref):
        t = dis_ref[:, :40] * (q0_ref[...] + q1_ref[...]) + b2_ref[...]
        m = jnp.max(t, axis=1, keepdims=True)
        e = t - m
        out_ref[...] = e - jnp.log(jnp.sum(jnp.exp(e), axis=1, keepdims=True))

    return pl.pallas_call(
        body,
        grid=(GRID,),
        in_specs=[
            pl.BlockSpec((BLK, 40), lambda i: (i, 0)),
            pl.BlockSpec((BLK, 40), lambda i: (i, 0)),
            pl.BlockSpec((BLK, 128), lambda i: (i, 0)),
            pl.BlockSpec((1, 40), lambda i: (0, 0)),
        ],
        out_specs=pl.BlockSpec((BLK, 40), lambda i: (i, 0)),
        out_shape=jax.ShapeDtypeStruct((NPAD, 40), jnp.float32),
    )(q0, q1, disb, b2)


def kernel(x, edge_index, W1, b1, W2, b2):
    ei = edge_index.astype(jnp.int32)
    E = ei.shape[1]
    nchunks = -(-E // (CH * NW * 10)) * NW * 10
    EPAD = nchunks * CH
    pad = EPAD - E
    padv = jnp.full((pad,), DUMMY, jnp.int32)
    src = jnp.concatenate([ei[0], padv]).reshape(-1, CH)
    dst = jnp.concatenate([ei[1], padv]).reshape(-1, CH)

    xp = jnp.pad(x, ((0, NPAD - N), (0, 0)))
    z16 = jnp.zeros((NPAD, DEGW), jnp.float32)
    z40 = jnp.zeros((NPAD, 40), jnp.float32)
    ones16 = jnp.ones((CH, DEGW), jnp.float32)

    degp = _make_deg(nchunks // NW)(dst, z16, ones16)
    y, disb = _t1(degp[0], degp[1], xp)
    p = _make_agg1(nchunks // NS)(y, src, dst)
    y2 = _t2(p, disb, W1, b1.reshape(1, -1), W2)
    q = _make_agg2(nchunks // NW)(y2, src, dst, z40)
    out = _t3(q[0], q[1], disb, b2.reshape(1, -1))
    return out[:N]
